# baseline (speedup 1.0000x reference)
import jax
import jax.numpy as jnp
from jax import lax
from jax.experimental import pallas as pl
from jax.experimental.pallas import tpu as pltpu
from jax.experimental.pallas import tpu_sc as plsc

R = 128
C = 32768
NC = 2
NS = 16
L = 16
RPW = 2
SC_ROWS = NC * NS * RPW
TC_ROWS = R - SC_ROWS
K = 8
STEPS = C // L
GROUPS = STEPS // K
TCB = 16

_BIG = 2**30


def _shuffle(x, d):
    perm = lax.iota(jnp.int32, L) ^ d
    return x.at[perm].get(mode="promise_in_bounds")


NCH = 4
H = C // NCH
HGROUPS = H // (K * L)


def _fresh_carry():
    neg_inf = jnp.full((L,), -jnp.inf, jnp.float32)
    zero = jnp.zeros((L,), jnp.int32)
    return (neg_inf,) * K, (zero,) * K, zero


def _scan_half(buf, off, carry):
    def body(g, carry):
        ms, gs, gvec = carry
        base = off + g * (K * L)
        new_ms, new_gs = [], []
        for k in range(K):
            v = buf[pl.ds(base + k * L, L)]
            p = v > ms[k]
            new_ms.append(jnp.maximum(ms[k], v))
            new_gs.append(jnp.where(p, gvec, gs[k]))
        return tuple(new_ms), tuple(new_gs), gvec + 1

    return lax.fori_loop(0, HGROUPS, body, carry)


def _finish_row(carry):
    ms, gs, _ = carry
    iota = lax.iota(jnp.int32, L)
    m = ms[0]
    for k in range(1, K):
        m = jnp.maximum(m, ms[k])
    for d in (1, 2, 4, 8):
        m = jnp.maximum(m, _shuffle(m, d))
    cand = jnp.full((L,), _BIG, jnp.int32)
    for k in range(K):
        idx_k = lax.bitwise_or(lax.shift_left(gs[k], 7), iota + k * L)
        cand = jnp.minimum(cand, jnp.where(ms[k] == m, idx_k, _BIG))
    for d in (1, 2, 4, 8):
        cand = jnp.minimum(cand, _shuffle(cand, d))
    return cand


def _sc_body(x_hbm, out_hbm, buf, res_v, sem0, sem1, sem2, sem3, sem4, sem5, sem6, sem7):
    c = lax.axis_index("c")
    s = lax.axis_index("s")
    wid = c * NS + s
    base_row = wid * RPW
    sems = (sem0, sem1, sem2, sem3, sem4, sem5, sem6, sem7)
    nchunks = NCH * RPW

    def chunk_src(i):
        return x_hbm.at[base_row + i // NCH, pl.ds((i % NCH) * H, H)]

    copies = [pltpu.async_copy(chunk_src(0), buf.at[pl.ds(0, H)], sems[0])]
    for i in range(nchunks):
        if i + 1 < nchunks:
            copies.append(
                pltpu.async_copy(
                    chunk_src(i + 1), buf.at[pl.ds((i + 1) * H, H)], sems[i + 1]
                )
            )
        copies[i].wait()
        if i % NCH == 0:
            carry = _fresh_carry()
        carry = _scan_half(buf, i * H, carry)
        if i % NCH == NCH - 1:
            res_v[...] = _finish_row(carry).astype(jnp.float32)
            pltpu.sync_copy(res_v, out_hbm.at[base_row + i // NCH])


_sc_argmax = pl.kernel(
    _sc_body,
    out_type=jax.ShapeDtypeStruct((SC_ROWS, L), jnp.float32),
    mesh=plsc.VectorSubcoreMesh(core_axis_name="c", subcore_axis_name="s"),
    scratch_types=[
        pltpu.VMEM((2 * C,), jnp.float32),
        pltpu.VMEM((L,), jnp.float32),
        pltpu.SemaphoreType.DMA,
        pltpu.SemaphoreType.DMA,
        pltpu.SemaphoreType.DMA,
        pltpu.SemaphoreType.DMA,
        pltpu.SemaphoreType.DMA,
        pltpu.SemaphoreType.DMA,
        pltpu.SemaphoreType.DMA,
        pltpu.SemaphoreType.DMA,
    ],
)


KT = 8


def _tc_body(x_ref, o_ref):
    def group(g, carry):
        ms, gs, gvec = carry
        new_ms, new_gs = [], []
        for k in range(KT):
            v = x_ref[:, pl.ds((g * KT + k) * 128, 128)]
            p = v > ms[k]
            new_ms.append(jnp.maximum(ms[k], v))
            new_gs.append(jnp.where(p, gvec, gs[k]))
        return tuple(new_ms), tuple(new_gs), gvec + 1

    m0 = jnp.full((TCB, 128), -jnp.inf, jnp.float32)
    z = jnp.zeros((TCB, 128), jnp.int32)
    ms, gs, _ = lax.fori_loop(
        0, C // 128 // KT, group, ((m0,) * KT, (z,) * KT, z)
    )
    m = ms[0]
    for k in range(1, KT):
        m = jnp.maximum(m, ms[k])
    mm = jnp.max(m, axis=1, keepdims=True)
    lane = jax.lax.broadcasted_iota(jnp.int32, (TCB, 128), 1)
    cand = jnp.full((TCB, 128), _BIG, jnp.int32)
    for k in range(KT):
        idx_k = lax.bitwise_or(lax.shift_left(gs[k] * KT + k, 7), lane)
        cand = jnp.minimum(cand, jnp.where(ms[k] == mm, idx_k, _BIG))
    best = jnp.min(cand, axis=1, keepdims=True)
    o_ref[...] = jnp.broadcast_to(best, (TCB, 128)).astype(jnp.float32)


def _make_tc(row0, nrows):
    return pl.pallas_call(
        _tc_body,
        grid=(nrows // TCB,),
        in_specs=[pl.BlockSpec((TCB, C), lambda i, r0=row0: (i + r0 // TCB, 0))],
        out_specs=pl.BlockSpec((TCB, 128), lambda i: (i, 0)),
        out_shape=jax.ShapeDtypeStruct((nrows, 128), jnp.float32),
    )


_tc_argmax = _make_tc(SC_ROWS, TC_ROWS)


def kernel(x):
    y = _sc_argmax(x)
    tc = _tc_argmax(x)
    return jnp.concatenate([y[:, :1], tc[:, :1]], axis=0)


# speedup vs baseline: 1.0242x; 1.0242x over previous
import jax
import jax.numpy as jnp
from jax import lax
from jax.experimental import pallas as pl
from jax.experimental.pallas import tpu as pltpu
from jax.experimental.pallas import tpu_sc as plsc

R = 128
C = 32768
NC = 2
NS = 16
L = 16
RPW = 2
SC_ROWS = NC * NS * RPW
TC_ROWS = R - SC_ROWS
K = 8
STEPS = C // L
GROUPS = STEPS // K
TCB = 16

_BIG = 2**30


def _shuffle(x, d):
    perm = lax.iota(jnp.int32, L) ^ d
    return x.at[perm].get(mode="promise_in_bounds")


H = C // 2
HGROUPS = H // (K * L)


def _fresh_carry():
    neg_inf = jnp.full((L,), -jnp.inf, jnp.float32)
    zero = jnp.zeros((L,), jnp.int32)
    return (neg_inf,) * K, (zero,) * K, zero


def _scan_half(buf, off, carry):
    def body(g, carry):
        ms, gs, gvec = carry
        base = off + g * (K * L)
        new_ms, new_gs = [], []
        for k in range(K):
            v = buf[pl.ds(base + k * L, L)]
            p = v > ms[k]
            new_ms.append(jnp.maximum(ms[k], v))
            new_gs.append(jnp.where(p, gvec, gs[k]))
        return tuple(new_ms), tuple(new_gs), gvec + 1

    return lax.fori_loop(0, HGROUPS, body, carry)


def _finish_row(carry):
    ms, gs, _ = carry
    iota = lax.iota(jnp.int32, L)
    m = ms[0]
    for k in range(1, K):
        m = jnp.maximum(m, ms[k])
    for d in (1, 2, 4, 8):
        m = jnp.maximum(m, _shuffle(m, d))
    cand = jnp.full((L,), _BIG, jnp.int32)
    for k in range(K):
        idx_k = lax.bitwise_or(lax.shift_left(gs[k], 7), iota + k * L)
        cand = jnp.minimum(cand, jnp.where(ms[k] == m, idx_k, _BIG))
    for d in (1, 2, 4, 8):
        cand = jnp.minimum(cand, _shuffle(cand, d))
    return cand


def _sc_body(x_hbm, out_hbm, buf, res_v, sem0, sem1, sem2, sem3):
    c = lax.axis_index("c")
    s = lax.axis_index("s")
    wid = c * NS + s
    base_row = wid * RPW
    sems = (sem0, sem1, sem2, sem3)
    nchunks = 2 * RPW

    def chunk_src(i):
        return x_hbm.at[base_row + i // 2, pl.ds((i % 2) * H, H)]

    copies = [pltpu.async_copy(chunk_src(0), buf.at[pl.ds(0, H)], sems[0])]
    for i in range(nchunks):
        if i + 1 < nchunks:
            copies.append(
                pltpu.async_copy(
                    chunk_src(i + 1), buf.at[pl.ds((i + 1) * H, H)], sems[i + 1]
                )
            )
        copies[i].wait()
        if i % 2 == 0:
            carry = _fresh_carry()
        carry = _scan_half(buf, i * H, carry)
        if i % 2 == 1:
            res_v[...] = _finish_row(carry).astype(jnp.float32)
            pltpu.sync_copy(res_v, out_hbm.at[base_row + i // 2])


_sc_argmax = pl.kernel(
    _sc_body,
    out_type=jax.ShapeDtypeStruct((SC_ROWS, L), jnp.float32),
    mesh=plsc.VectorSubcoreMesh(core_axis_name="c", subcore_axis_name="s"),
    scratch_types=[
        pltpu.VMEM((2 * C,), jnp.float32),
        pltpu.VMEM((L,), jnp.float32),
        pltpu.SemaphoreType.DMA,
        pltpu.SemaphoreType.DMA,
        pltpu.SemaphoreType.DMA,
        pltpu.SemaphoreType.DMA,
    ],
)


KT = 8


def _tc_body(x_ref, o_ref):
    def group(g, carry):
        ms, gs, gvec = carry
        new_ms, new_gs = [], []
        for k in range(KT):
            v = x_ref[:, pl.ds((g * KT + k) * 128, 128)]
            p = v > ms[k]
            new_ms.append(jnp.maximum(ms[k], v))
            new_gs.append(jnp.where(p, gvec, gs[k]))
        return tuple(new_ms), tuple(new_gs), gvec + 1

    m0 = jnp.full((TCB, 128), -jnp.inf, jnp.float32)
    z = jnp.zeros((TCB, 128), jnp.int32)
    ms, gs, _ = lax.fori_loop(
        0, C // 128 // KT, group, ((m0,) * KT, (z,) * KT, z)
    )
    m = ms[0]
    for k in range(1, KT):
        m = jnp.maximum(m, ms[k])
    mm = jnp.max(m, axis=1, keepdims=True)
    lane = jax.lax.broadcasted_iota(jnp.int32, (TCB, 128), 1)
    cand = jnp.full((TCB, 128), _BIG, jnp.int32)
    for k in range(KT):
        idx_k = lax.bitwise_or(lax.shift_left(gs[k] * KT + k, 7), lane)
        cand = jnp.minimum(cand, jnp.where(ms[k] == mm, idx_k, _BIG))
    best = jnp.min(cand, axis=1, keepdims=True)
    o_ref[...] = jnp.broadcast_to(best, (TCB, 128)).astype(jnp.float32)


def _make_tc(row0, nrows):
    return pl.pallas_call(
        _tc_body,
        grid=(nrows // TCB,),
        in_specs=[pl.BlockSpec((TCB, C), lambda i, r0=row0: (i + r0 // TCB, 0))],
        out_specs=pl.BlockSpec((TCB, 128), lambda i: (i, 0)),
        out_shape=jax.ShapeDtypeStruct((nrows, 128), jnp.float32),
    )


_tc_argmax = _make_tc(SC_ROWS, TC_ROWS)


def kernel(x):
    y = _sc_argmax(x)
    tc = _tc_argmax(x)
    return jnp.concatenate([y[:, :1], tc[:, :1]], axis=0)
